# baseline (speedup 1.0000x reference)
import functools

import jax
import jax.numpy as jnp
from jax import lax
from jax.experimental import pallas as pl
from jax.experimental.pallas import tpu as pltpu
from jax.experimental.pallas import tpu_sc as plsc

_NUM_CORES = 2
_NUM_SUBCORES = 16
_NUM_WORKERS = _NUM_CORES * _NUM_SUBCORES
_CHUNK = 128


def _gather_sc(idx3, table):
    nw, n_chunks, chunk = idx3.shape
    _, d = table.shape
    b_per_w = n_chunks * chunk

    mesh = plsc.VectorSubcoreMesh(
        core_axis_name="c",
        subcore_axis_name="s",
        num_cores=_NUM_CORES,
        num_subcores=_NUM_SUBCORES,
    )

    @functools.partial(
        pl.kernel,
        out_type=jax.ShapeDtypeStruct((nw * b_per_w, d), jnp.float32),
        mesh=mesh,
        scratch_types=[
            pltpu.VMEM((n_chunks, chunk), jnp.int32),
            pltpu.VMEM((b_per_w, d), jnp.float32),
            pltpu.SemaphoreType.DMA,
        ],
        compiler_params=pltpu.CompilerParams(use_tc_tiling_on_sc=False),
    )
    def k(idx_hbm, table_hbm, out_hbm, idx_v, rows_v, sem):
        wid = lax.axis_index("s") * _NUM_CORES + lax.axis_index("c")
        pltpu.sync_copy(idx_hbm.at[wid], idx_v)
        copies = []
        for j in range(n_chunks):
            copies.append(
                pltpu.async_copy(
                    table_hbm.at[idx_v.at[j]],
                    rows_v.at[pl.ds(j * chunk, chunk)],
                    sem,
                )
            )
        for c in copies:
            c.wait()
        pltpu.sync_copy(rows_v, out_hbm.at[pl.ds(wid * b_per_w, b_per_w)])

    return k(idx3, table)


def kernel(inputs, length, mode, table):
    b = inputs.shape[0]
    assert b % (_NUM_WORKERS * _CHUNK) == 0, b
    n_chunks = b // (_NUM_WORKERS * _CHUNK)
    idx3 = inputs.reshape(_NUM_WORKERS, n_chunks, _CHUNK)
    return _gather_sc(idx3, table)


# speedup vs baseline: 1.7096x; 1.7096x over previous
import functools

import jax
import jax.numpy as jnp
from jax import lax
from jax.experimental import pallas as pl
from jax.experimental.pallas import tpu as pltpu
from jax.experimental.pallas import tpu_sc as plsc

_NUM_CORES = 2
_NUM_SUBCORES = 16
_NUM_WORKERS = _NUM_CORES * _NUM_SUBCORES
_LANES = 16


def _gather_sc(idx2, table):
    nw, b_per_w = idx2.shape
    _, d = table.shape

    mesh = plsc.VectorSubcoreMesh(
        core_axis_name="c",
        subcore_axis_name="s",
        num_cores=_NUM_CORES,
        num_subcores=_NUM_SUBCORES,
    )

    @functools.partial(
        pl.kernel,
        out_type=jax.ShapeDtypeStruct((nw * b_per_w, d), jnp.float32),
        mesh=mesh,
        scratch_types=[
            pltpu.VMEM((b_per_w,), jnp.int32),
            pltpu.VMEM((b_per_w, d), jnp.float32),
            pltpu.SemaphoreType.DMA,
            pltpu.SemaphoreType.DMA,
        ],
        compiler_params=pltpu.CompilerParams(use_tc_tiling_on_sc=True),
    )
    def k(idx_hbm, table_hbm, out_hbm, idx_v, rows_v, sem_i, sem):
        wid = lax.axis_index("s") * _NUM_CORES + lax.axis_index("c")
        pltpu.async_copy(idx_hbm.at[wid], idx_v, sem_i).wait()

        def body(g, _):
            vec = idx_v[pl.ds(g * _LANES, _LANES)]
            for lane in range(_LANES):
                row = vec[lane]
                pltpu.async_copy(table_hbm.at[row], rows_v.at[g * _LANES + lane], sem)
            return 0

        lax.fori_loop(0, b_per_w // _LANES, body, 0)
        pltpu.make_async_copy(out_hbm.at[pl.ds(0, b_per_w)], rows_v, sem).wait()
        pltpu.sync_copy(rows_v, out_hbm.at[pl.ds(wid * b_per_w, b_per_w)])

    return k(idx2, table)


def kernel(inputs, length, mode, table):
    b = inputs.shape[0]
    assert b % _NUM_WORKERS == 0, b
    idx2 = inputs.reshape(_NUM_WORKERS, b // _NUM_WORKERS)
    return _gather_sc(idx2, table)
